# baseline (speedup 1.0000x reference)
import functools

import jax
import jax.numpy as jnp
from jax import lax
from jax.experimental import pallas as pl
from jax.experimental.pallas import tpu as pltpu
from jax.experimental.pallas import tpu_sc as plsc

N = 1024
MAXN = 80
L = 16
NC, NS = 2, 16
NW = NC * NS
ROWS = N // NW
GR = 8
NCHUNK = N // L
BUF = 128
BIG = 0x3FFFFFFF
SIDE = 25.0
HALF = 12.5
CUT2 = 36.0
TCB = 64


def _tc_body(posq_ref, posmat_ref, out_ref):
    shape = (TCB, 8, 128)
    qx = posq_ref[:, 0:1].reshape(TCB, 1, 1)
    qy = posq_ref[:, 1:2].reshape(TCB, 1, 1)
    qz = posq_ref[:, 2:3].reshape(TCB, 1, 1)
    px = posmat_ref[0][None]
    py = posmat_ref[1][None]
    pz = posmat_ref[2][None]
    dx = px - qx
    dy = py - qy
    dz = pz - qz
    gx, lx = dx > HALF, dx < -HALF
    gy, ly = dy > HALF, dy < -HALF
    gz, lz = dz > HALF, dz < -HALF
    wx = (px + jnp.where(gx, -SIDE, jnp.where(lx, SIDE, 0.0))) - qx
    wy = (py + jnp.where(gy, -SIDE, jnp.where(ly, SIDE, 0.0))) - qy
    wz = (pz + jnp.where(gz, -SIDE, jnp.where(lz, SIDE, 0.0))) - qz
    d2 = (wx * wx + wy * wy) + wz * wz
    qid = (pl.program_id(0) * TCB
           + lax.broadcasted_iota(jnp.int32, shape, 0))
    pid = (lax.broadcasted_iota(jnp.int32, shape, 1) * 128
           + lax.broadcasted_iota(jnp.int32, shape, 2))
    hit = (d2 <= CUT2) & (qid != pid)
    key = (13 * N + pid
           + jnp.where(gx, -N, 0) + jnp.where(lx, N, 0)
           + jnp.where(gy, -3 * N, 0) + jnp.where(ly, 3 * N, 0)
           + jnp.where(gz, -9 * N, 0) + jnp.where(lz, 9 * N, 0))
    out_ref[...] = jnp.where(hit, key, BIG).reshape(TCB * 8, 128)


def _tc_keys(posq, posmat):
    return pl.pallas_call(
        _tc_body,
        grid=(N // TCB,),
        in_specs=[
            pl.BlockSpec((TCB, 3), lambda i: (i, 0)),
            pl.BlockSpec((3, 8, 128), lambda i: (0, 0, 0)),
        ],
        out_specs=pl.BlockSpec((TCB * 8, 128), lambda i: (i, 0)),
        out_shape=jax.ShapeDtypeStruct((N * 8, 128), jnp.int32),
    )(posq, posmat)


def _merge(a, b, need=None):
    s = list(a) + [lax.rev(v, (0,)) for v in reversed(list(b))]
    n = len(s)
    stride = n // 2
    while stride >= 1:
        for base in range(0, n, 2 * stride):
            for i in range(base, base + stride):
                lo = jnp.minimum(s[i], s[i + stride])
                hi = jnp.maximum(s[i], s[i + stride])
                s[i], s[i + stride] = lo, hi
        stride //= 2
    return [jnp.sort(v) if (need is None or i < need) else v
            for i, v in enumerate(s)]


def _sort_vregs(vs, need=None):
    runs = [[jnp.sort(v)] for v in vs]
    while len(runs) > 1:
        last = len(runs) == 2
        runs = [_merge(runs[i], runs[i + 1], need=need if last else None)
                for i in range(0, len(runs), 2)]
    return runs[0]


_mesh = plsc.VectorSubcoreMesh(core_axis_name="c", subcore_axis_name="s")


@functools.partial(
    pl.kernel,
    out_type=(
        jax.ShapeDtypeStruct((N * MAXN,), jnp.int32),
        jax.ShapeDtypeStruct((N * MAXN,), jnp.int32),
        jax.ShapeDtypeStruct((N * MAXN,), jnp.int32),
        jax.ShapeDtypeStruct((N * MAXN,), jnp.int32),
        jax.ShapeDtypeStruct((NW, L), jnp.int32),
    ),
    mesh=_mesh,
    compiler_params=pltpu.CompilerParams(needs_layout_passes=False),
    scratch_types=[
        pltpu.VMEM((GR * N,), jnp.int32),
        pltpu.VMEM((N + L,), jnp.int32),
        pltpu.VMEM((ROWS * MAXN,), jnp.int32),
        pltpu.VMEM((ROWS * MAXN,), jnp.int32),
        pltpu.VMEM((ROWS * MAXN,), jnp.int32),
        pltpu.VMEM((ROWS * MAXN,), jnp.int32),
        pltpu.VMEM((L,), jnp.int32),
    ],
)
def _sc_kernel(keys_hbm, nbr_hbm, cx_hbm, cy_hbm, cz_hbm, max_hbm,
               kb_v, keys_v, nbr_v, cx_v, cy_v, cz_v, max_v):
    wid = lax.axis_index("s") * NC + lax.axis_index("c")
    base_row = wid * ROWS
    bigv = jnp.full((L,), BIG, jnp.int32)

    def row_body(r, maxcnt):
        @pl.when(r % GR == 0)
        def _fetch():
            pltpu.sync_copy(keys_hbm.at[pl.ds((base_row + r) * N, GR * N)],
                            kb_v)

        rb = (r % GR) * N
        for j in range(BUF // L):
            keys_v[pl.ds(j * L, L)] = bigv

        def chunk_body(i, cnt):
            kv = kb_v[pl.ds(rb + i * L, L)]
            hit = kv < BIG
            inc = hit.astype(jnp.int32)
            pre = plsc.cumsum(inc)
            idx = cnt + pre - 1
            plsc.store_scatter(keys_v, [idx], kv, mask=hit)
            return cnt + plsc.all_reduce_population_count(hit)

        cnt = lax.fori_loop(0, NCHUNK, chunk_body, jnp.zeros((L,), jnp.int32),
                            unroll=4)

        vs = [keys_v[pl.ds(j * L, L)] for j in range(BUF // L)]
        svs = _sort_vregs(vs, need=MAXN // L)
        for j in range(MAXN // L):
            k = svs[j]
            pad = k >= jnp.int32(27 * N)
            p = jnp.where(pad, -1, k & (N - 1))
            c = k >> 10
            ob = r * MAXN + j * L
            nbr_v[pl.ds(ob, L)] = p
            cx_v[pl.ds(ob, L)] = jnp.where(pad, 1, c % 3 - 1)
            cy_v[pl.ds(ob, L)] = jnp.where(pad, 1, (c // 3) % 3 - 1)
            cz_v[pl.ds(ob, L)] = jnp.where(pad, 1, c // 9 - 1)
        return jnp.maximum(maxcnt, cnt)

    maxcnt = lax.fori_loop(0, ROWS, row_body, jnp.zeros((L,), jnp.int32),
                           unroll=2)
    max_v[...] = maxcnt
    span = pl.ds(wid * ROWS * MAXN, ROWS * MAXN)
    pltpu.sync_copy(nbr_v, nbr_hbm.at[span])
    pltpu.sync_copy(cx_v, cx_hbm.at[span])
    pltpu.sync_copy(cy_v, cy_hbm.at[span])
    pltpu.sync_copy(cz_v, cz_hbm.at[span])
    pltpu.sync_copy(max_v, max_hbm.at[wid])


def kernel(positions, max_neighbours):
    del max_neighbours
    pos = positions.astype(jnp.float32)
    posmat = pos.T.reshape(3, 8, 128)
    keys = _tc_keys(pos, posmat)
    nbr_flat, cx_flat, cy_flat, cz_flat, maxs = _sc_kernel(
        keys.reshape(N * N))
    neighbours = nbr_flat.reshape(N, MAXN)
    cells = jnp.stack(
        [cx_flat.reshape(N, MAXN), cy_flat.reshape(N, MAXN),
         cz_flat.reshape(N, MAXN)], axis=-1)
    return neighbours, cells, jnp.max(maxs)


# speedup vs baseline: 1.2826x; 1.2826x over previous
import functools

import jax
import jax.numpy as jnp
from jax import lax
from jax.experimental import pallas as pl
from jax.experimental.pallas import tpu as pltpu
from jax.experimental.pallas import tpu_sc as plsc

N = 1024
MAXN = 80
L = 16
NC, NS = 2, 16
NW = NC * NS
ROWS = N // NW
GR = 8
NCHUNK = N // L
BUF = 128
BIG = 0x3FFFFFFF
SIDE = 25.0
HALF = 12.5
CUT2 = 36.0
TCB = 32


def _tc_body(posq_ref, posmat_ref, out_ref):
    shape = (TCB, 8, 128)
    qx = posq_ref[:, 0:1].reshape(TCB, 1, 1)
    qy = posq_ref[:, 1:2].reshape(TCB, 1, 1)
    qz = posq_ref[:, 2:3].reshape(TCB, 1, 1)
    px = posmat_ref[0][None]
    py = posmat_ref[1][None]
    pz = posmat_ref[2][None]
    dx = px - qx
    dy = py - qy
    dz = pz - qz
    gx, lx = dx > HALF, dx < -HALF
    gy, ly = dy > HALF, dy < -HALF
    gz, lz = dz > HALF, dz < -HALF
    wx = (px + jnp.where(gx, -SIDE, jnp.where(lx, SIDE, 0.0))) - qx
    wy = (py + jnp.where(gy, -SIDE, jnp.where(ly, SIDE, 0.0))) - qy
    wz = (pz + jnp.where(gz, -SIDE, jnp.where(lz, SIDE, 0.0))) - qz
    d2 = (wx * wx + wy * wy) + wz * wz
    qid = (pl.program_id(0) * TCB
           + lax.broadcasted_iota(jnp.int32, shape, 0))
    pid = (lax.broadcasted_iota(jnp.int32, shape, 1) * 128
           + lax.broadcasted_iota(jnp.int32, shape, 2))
    hit = (d2 <= CUT2) & (qid != pid)
    key = (13 * N + pid
           + jnp.where(gx, -N, 0) + jnp.where(lx, N, 0)
           + jnp.where(gy, -3 * N, 0) + jnp.where(ly, 3 * N, 0)
           + jnp.where(gz, -9 * N, 0) + jnp.where(lz, 9 * N, 0))
    out_ref[...] = jnp.where(hit, key, BIG).reshape(TCB * 8, 128)


def _tc_keys(posq, posmat):
    return pl.pallas_call(
        _tc_body,
        grid=(N // TCB,),
        in_specs=[
            pl.BlockSpec((TCB, 3), lambda i: (i, 0)),
            pl.BlockSpec((3, 8, 128), lambda i: (0, 0, 0)),
        ],
        out_specs=pl.BlockSpec((TCB * 8, 128), lambda i: (i, 0)),
        out_shape=jax.ShapeDtypeStruct((N * 8, 128), jnp.int32),
    )(posq, posmat)


def _merge(a, b, need=None):
    s = list(a) + [lax.rev(v, (0,)) for v in reversed(list(b))]
    n = len(s)
    stride = n // 2
    while stride >= 1:
        for base in range(0, n, 2 * stride):
            for i in range(base, base + stride):
                lo = jnp.minimum(s[i], s[i + stride])
                hi = jnp.maximum(s[i], s[i + stride])
                s[i], s[i + stride] = lo, hi
        stride //= 2
    return [jnp.sort(v) if (need is None or i < need) else v
            for i, v in enumerate(s)]


def _sort_vregs(vs, need=None):
    runs = [[jnp.sort(v)] for v in vs]
    while len(runs) > 1:
        last = len(runs) == 2
        runs = [_merge(runs[i], runs[i + 1], need=need if last else None)
                for i in range(0, len(runs), 2)]
    return runs[0]


_mesh = plsc.VectorSubcoreMesh(core_axis_name="c", subcore_axis_name="s")


@functools.partial(
    pl.kernel,
    out_type=(
        jax.ShapeDtypeStruct((N * MAXN,), jnp.int32),
        jax.ShapeDtypeStruct((N * MAXN,), jnp.int32),
        jax.ShapeDtypeStruct((N * MAXN,), jnp.int32),
        jax.ShapeDtypeStruct((N * MAXN,), jnp.int32),
        jax.ShapeDtypeStruct((NW, L), jnp.int32),
    ),
    mesh=_mesh,
    compiler_params=pltpu.CompilerParams(needs_layout_passes=False),
    scratch_types=[
        pltpu.VMEM((GR * N,), jnp.int32),
        pltpu.VMEM((N + L,), jnp.int32),
        pltpu.VMEM((ROWS * MAXN,), jnp.int32),
        pltpu.VMEM((ROWS * MAXN,), jnp.int32),
        pltpu.VMEM((ROWS * MAXN,), jnp.int32),
        pltpu.VMEM((ROWS * MAXN,), jnp.int32),
        pltpu.VMEM((L,), jnp.int32),
    ],
)
def _sc_kernel(keys_hbm, nbr_hbm, cx_hbm, cy_hbm, cz_hbm, max_hbm,
               kb_v, keys_v, nbr_v, cx_v, cy_v, cz_v, max_v):
    wid = lax.axis_index("s") * NC + lax.axis_index("c")
    base_row = wid * ROWS
    bigv = jnp.full((L,), BIG, jnp.int32)

    def row_body(r, maxcnt):
        @pl.when(r % GR == 0)
        def _fetch():
            pltpu.sync_copy(keys_hbm.at[pl.ds((base_row + r) * N, GR * N)],
                            kb_v)

        rb = (r % GR) * N
        for j in range(BUF // L):
            keys_v[pl.ds(j * L, L)] = bigv

        def chunk_body(i, cnt):
            kv = kb_v[pl.ds(rb + i * L, L)]
            hit = kv < BIG
            inc = hit.astype(jnp.int32)
            pre = plsc.cumsum(inc)
            idx = cnt + pre - 1
            plsc.store_scatter(keys_v, [idx], kv, mask=hit)
            return cnt + plsc.all_reduce_population_count(hit)

        cnt = lax.fori_loop(0, NCHUNK, chunk_body, jnp.zeros((L,), jnp.int32),
                            unroll=4)

        vs = [keys_v[pl.ds(j * L, L)] for j in range(BUF // L)]
        nout = MAXN // L

        def _small_sort():
            return _sort_vregs(vs[:4]) + [vs[4]]

        def _big_sort():
            return _sort_vregs(vs, need=nout)[:nout]

        svs = lax.cond(cnt[0] <= 4 * L, _small_sort, _big_sort)
        for j in range(MAXN // L):
            k = svs[j]
            pad = k >= jnp.int32(27 * N)
            p = jnp.where(pad, -1, k & (N - 1))
            c = k >> 10
            ob = r * MAXN + j * L
            nbr_v[pl.ds(ob, L)] = p
            cx_v[pl.ds(ob, L)] = jnp.where(pad, 1, c % 3 - 1)
            cy_v[pl.ds(ob, L)] = jnp.where(pad, 1, (c // 3) % 3 - 1)
            cz_v[pl.ds(ob, L)] = jnp.where(pad, 1, c // 9 - 1)
        return jnp.maximum(maxcnt, cnt)

    maxcnt = lax.fori_loop(0, ROWS, row_body, jnp.zeros((L,), jnp.int32))
    max_v[...] = maxcnt
    span = pl.ds(wid * ROWS * MAXN, ROWS * MAXN)
    pltpu.sync_copy(nbr_v, nbr_hbm.at[span])
    pltpu.sync_copy(cx_v, cx_hbm.at[span])
    pltpu.sync_copy(cy_v, cy_hbm.at[span])
    pltpu.sync_copy(cz_v, cz_hbm.at[span])
    pltpu.sync_copy(max_v, max_hbm.at[wid])


def kernel(positions, max_neighbours):
    del max_neighbours
    pos = positions.astype(jnp.float32)
    posmat = pos.T.reshape(3, 8, 128)
    keys = _tc_keys(pos, posmat)
    nbr_flat, cx_flat, cy_flat, cz_flat, maxs = _sc_kernel(
        keys.reshape(N * N))
    neighbours = nbr_flat.reshape(N, MAXN)
    cells = jnp.stack(
        [cx_flat.reshape(N, MAXN), cy_flat.reshape(N, MAXN),
         cz_flat.reshape(N, MAXN)], axis=-1)
    return neighbours, cells, jnp.max(maxs)


# speedup vs baseline: 1.2841x; 1.0012x over previous
import functools

import jax
import jax.numpy as jnp
from jax import lax
from jax.experimental import pallas as pl
from jax.experimental.pallas import tpu as pltpu
from jax.experimental.pallas import tpu_sc as plsc

N = 1024
MAXN = 80
L = 16
NC, NS = 2, 16
NW = NC * NS
ROWS = N // NW
GR = 8
NCHUNK = N // L
BUF = 128
BIG = 0x3FFFFFFF
SIDE = 25.0
HALF = 12.5
CUT2 = 36.0
TCB = 32


def _tc_body(posq_ref, posmat_ref, out_ref):
    shape = (TCB, 8, 128)
    qx = posq_ref[:, 0:1].reshape(TCB, 1, 1)
    qy = posq_ref[:, 1:2].reshape(TCB, 1, 1)
    qz = posq_ref[:, 2:3].reshape(TCB, 1, 1)
    px = posmat_ref[0][None]
    py = posmat_ref[1][None]
    pz = posmat_ref[2][None]
    dx = px - qx
    dy = py - qy
    dz = pz - qz
    gx, lx = dx > HALF, dx < -HALF
    gy, ly = dy > HALF, dy < -HALF
    gz, lz = dz > HALF, dz < -HALF
    wx = (px + jnp.where(gx, -SIDE, jnp.where(lx, SIDE, 0.0))) - qx
    wy = (py + jnp.where(gy, -SIDE, jnp.where(ly, SIDE, 0.0))) - qy
    wz = (pz + jnp.where(gz, -SIDE, jnp.where(lz, SIDE, 0.0))) - qz
    d2 = (wx * wx + wy * wy) + wz * wz
    qid = (pl.program_id(0) * TCB
           + lax.broadcasted_iota(jnp.int32, shape, 0))
    pid = (lax.broadcasted_iota(jnp.int32, shape, 1) * 128
           + lax.broadcasted_iota(jnp.int32, shape, 2))
    hit = (d2 <= CUT2) & (qid != pid)
    key = (13 * N + pid
           + jnp.where(gx, -N, 0) + jnp.where(lx, N, 0)
           + jnp.where(gy, -3 * N, 0) + jnp.where(ly, 3 * N, 0)
           + jnp.where(gz, -9 * N, 0) + jnp.where(lz, 9 * N, 0))
    out_ref[...] = jnp.where(hit, key, BIG).reshape(TCB * 8, 128)


def _tc_keys(posq, posmat):
    return pl.pallas_call(
        _tc_body,
        grid=(N // TCB,),
        in_specs=[
            pl.BlockSpec((TCB, 3), lambda i: (i, 0)),
            pl.BlockSpec((3, 8, 128), lambda i: (0, 0, 0)),
        ],
        out_specs=pl.BlockSpec((TCB * 8, 128), lambda i: (i, 0)),
        out_shape=jax.ShapeDtypeStruct((N * 8, 128), jnp.int32),
    )(posq, posmat)


def _merge(a, b, need=None):
    s = list(a) + [lax.rev(v, (0,)) for v in reversed(list(b))]
    n = len(s)
    stride = n // 2
    while stride >= 1:
        for base in range(0, n, 2 * stride):
            for i in range(base, base + stride):
                lo = jnp.minimum(s[i], s[i + stride])
                hi = jnp.maximum(s[i], s[i + stride])
                s[i], s[i + stride] = lo, hi
        stride //= 2
    return [jnp.sort(v) if (need is None or i < need) else v
            for i, v in enumerate(s)]


def _sort_vregs(vs, need=None):
    runs = [[jnp.sort(v)] for v in vs]
    while len(runs) > 1:
        last = len(runs) == 2
        runs = [_merge(runs[i], runs[i + 1], need=need if last else None)
                for i in range(0, len(runs), 2)]
    return runs[0]


_mesh = plsc.VectorSubcoreMesh(core_axis_name="c", subcore_axis_name="s")


@functools.partial(
    pl.kernel,
    out_type=(
        jax.ShapeDtypeStruct((N * MAXN,), jnp.int32),
        jax.ShapeDtypeStruct((N * MAXN,), jnp.int32),
        jax.ShapeDtypeStruct((N * MAXN,), jnp.int32),
        jax.ShapeDtypeStruct((N * MAXN,), jnp.int32),
        jax.ShapeDtypeStruct((NW, L), jnp.int32),
    ),
    mesh=_mesh,
    compiler_params=pltpu.CompilerParams(needs_layout_passes=False),
    scratch_types=[
        pltpu.VMEM((GR * N,), jnp.int32),
        pltpu.VMEM((N + L,), jnp.int32),
        pltpu.VMEM((ROWS * MAXN,), jnp.int32),
        pltpu.VMEM((ROWS * MAXN,), jnp.int32),
        pltpu.VMEM((ROWS * MAXN,), jnp.int32),
        pltpu.VMEM((ROWS * MAXN,), jnp.int32),
        pltpu.VMEM((L,), jnp.int32),
    ],
)
def _sc_kernel(keys_hbm, nbr_hbm, cx_hbm, cy_hbm, cz_hbm, max_hbm,
               kb_v, keys_v, nbr_v, cx_v, cy_v, cz_v, max_v):
    wid = lax.axis_index("s") * NC + lax.axis_index("c")
    base_row = wid * ROWS
    bigv = jnp.full((L,), BIG, jnp.int32)

    def row_body(r, maxcnt):
        @pl.when(r % GR == 0)
        def _fetch():
            pltpu.sync_copy(keys_hbm.at[pl.ds((base_row + r) * N, GR * N)],
                            kb_v)

        rb = (r % GR) * N
        for j in range(BUF // L):
            keys_v[pl.ds(j * L, L)] = bigv

        def chunk_body(i, cnt):
            kv = kb_v[pl.ds(rb + i * L, L)]
            hit = kv < BIG
            inc = hit.astype(jnp.int32)
            pre = plsc.cumsum(inc)
            idx = cnt + pre - 1
            plsc.store_scatter(keys_v, [idx], kv, mask=hit)
            return cnt + plsc.all_reduce_population_count(hit)

        cnt = lax.fori_loop(0, NCHUNK, chunk_body, jnp.zeros((L,), jnp.int32),
                            unroll=2)

        vs = [keys_v[pl.ds(j * L, L)] for j in range(BUF // L)]
        svs = _sort_vregs(vs, need=MAXN // L)
        for j in range(MAXN // L):
            k = svs[j]
            pad = k >= jnp.int32(27 * N)
            p = jnp.where(pad, -1, k & (N - 1))
            c = k >> 10
            ob = r * MAXN + j * L
            nbr_v[pl.ds(ob, L)] = p
            cx_v[pl.ds(ob, L)] = jnp.where(pad, 1, c % 3 - 1)
            cy_v[pl.ds(ob, L)] = jnp.where(pad, 1, (c // 3) % 3 - 1)
            cz_v[pl.ds(ob, L)] = jnp.where(pad, 1, c // 9 - 1)
        return jnp.maximum(maxcnt, cnt)

    maxcnt = lax.fori_loop(0, ROWS, row_body, jnp.zeros((L,), jnp.int32))
    max_v[...] = maxcnt
    span = pl.ds(wid * ROWS * MAXN, ROWS * MAXN)
    pltpu.sync_copy(nbr_v, nbr_hbm.at[span])
    pltpu.sync_copy(cx_v, cx_hbm.at[span])
    pltpu.sync_copy(cy_v, cy_hbm.at[span])
    pltpu.sync_copy(cz_v, cz_hbm.at[span])
    pltpu.sync_copy(max_v, max_hbm.at[wid])


def kernel(positions, max_neighbours):
    del max_neighbours
    pos = positions.astype(jnp.float32)
    posmat = pos.T.reshape(3, 8, 128)
    keys = _tc_keys(pos, posmat)
    nbr_flat, cx_flat, cy_flat, cz_flat, maxs = _sc_kernel(
        keys.reshape(N * N))
    neighbours = nbr_flat.reshape(N, MAXN)
    cells = jnp.stack(
        [cx_flat.reshape(N, MAXN), cy_flat.reshape(N, MAXN),
         cz_flat.reshape(N, MAXN)], axis=-1)
    return neighbours, cells, jnp.max(maxs)


# speedup vs baseline: 1.2863x; 1.0017x over previous
import functools

import jax
import jax.numpy as jnp
from jax import lax
from jax.experimental import pallas as pl
from jax.experimental.pallas import tpu as pltpu
from jax.experimental.pallas import tpu_sc as plsc

N = 1024
MAXN = 80
L = 16
NC, NS = 2, 16
NW = NC * NS
ROWS = N // NW
GR = 8
NCHUNK = N // L
BUF = 128
BIG = 0x3FFFFFFF
SIDE = 25.0
HALF = 12.5
CUT2 = 36.0
TCB = 32


def _tc_body(posq_ref, posmat_ref, out_ref):
    shape = (TCB, 8, 128)
    qx = posq_ref[:, 0:1].reshape(TCB, 1, 1)
    qy = posq_ref[:, 1:2].reshape(TCB, 1, 1)
    qz = posq_ref[:, 2:3].reshape(TCB, 1, 1)
    px = posmat_ref[0][None]
    py = posmat_ref[1][None]
    pz = posmat_ref[2][None]
    dx = px - qx
    dy = py - qy
    dz = pz - qz
    gx, lx = dx > HALF, dx < -HALF
    gy, ly = dy > HALF, dy < -HALF
    gz, lz = dz > HALF, dz < -HALF
    wx = (px + jnp.where(gx, -SIDE, jnp.where(lx, SIDE, 0.0))) - qx
    wy = (py + jnp.where(gy, -SIDE, jnp.where(ly, SIDE, 0.0))) - qy
    wz = (pz + jnp.where(gz, -SIDE, jnp.where(lz, SIDE, 0.0))) - qz
    d2 = (wx * wx + wy * wy) + wz * wz
    qid = (pl.program_id(0) * TCB
           + lax.broadcasted_iota(jnp.int32, shape, 0))
    pid = (lax.broadcasted_iota(jnp.int32, shape, 1) * 128
           + lax.broadcasted_iota(jnp.int32, shape, 2))
    hit = (d2 <= CUT2) & (qid != pid)
    key = (13 * N + pid
           + jnp.where(gx, -N, jnp.where(lx, N, 0))
           + jnp.where(gy, -3 * N, jnp.where(ly, 3 * N, 0))
           + jnp.where(gz, -9 * N, jnp.where(lz, 9 * N, 0)))
    out_ref[...] = jnp.where(hit, key, BIG).reshape(TCB * 8, 128)


def _tc_keys(posq, posmat):
    return pl.pallas_call(
        _tc_body,
        grid=(N // TCB,),
        in_specs=[
            pl.BlockSpec((TCB, 3), lambda i: (i, 0)),
            pl.BlockSpec((3, 8, 128), lambda i: (0, 0, 0)),
        ],
        out_specs=pl.BlockSpec((TCB * 8, 128), lambda i: (i, 0)),
        out_shape=jax.ShapeDtypeStruct((N * 8, 128), jnp.int32),
    )(posq, posmat)


def _merge(a, b, need=None):
    s = list(a) + [lax.rev(v, (0,)) for v in reversed(list(b))]
    n = len(s)
    stride = n // 2
    while stride >= 1:
        for base in range(0, n, 2 * stride):
            for i in range(base, base + stride):
                lo = jnp.minimum(s[i], s[i + stride])
                hi = jnp.maximum(s[i], s[i + stride])
                s[i], s[i + stride] = lo, hi
        stride //= 2
    return [jnp.sort(v) if (need is None or i < need) else v
            for i, v in enumerate(s)]


def _sort_vregs(vs, need=None):
    runs = [[jnp.sort(v)] for v in vs]
    while len(runs) > 1:
        last = len(runs) == 2
        runs = [_merge(runs[i], runs[i + 1], need=need if last else None)
                for i in range(0, len(runs), 2)]
    return runs[0]


_mesh = plsc.VectorSubcoreMesh(core_axis_name="c", subcore_axis_name="s")


@functools.partial(
    pl.kernel,
    out_type=(
        jax.ShapeDtypeStruct((N * MAXN,), jnp.int32),
        jax.ShapeDtypeStruct((N * MAXN,), jnp.int32),
        jax.ShapeDtypeStruct((N * MAXN,), jnp.int32),
        jax.ShapeDtypeStruct((N * MAXN,), jnp.int32),
        jax.ShapeDtypeStruct((NW, L), jnp.int32),
    ),
    mesh=_mesh,
    compiler_params=pltpu.CompilerParams(needs_layout_passes=False),
    scratch_types=[
        pltpu.VMEM((GR * N,), jnp.int32),
        pltpu.VMEM((N + L,), jnp.int32),
        pltpu.VMEM((ROWS * MAXN,), jnp.int32),
        pltpu.VMEM((ROWS * MAXN,), jnp.int32),
        pltpu.VMEM((ROWS * MAXN,), jnp.int32),
        pltpu.VMEM((ROWS * MAXN,), jnp.int32),
        pltpu.VMEM((L,), jnp.int32),
    ],
)
def _sc_kernel(keys_hbm, nbr_hbm, cx_hbm, cy_hbm, cz_hbm, max_hbm,
               kb_v, keys_v, nbr_v, cx_v, cy_v, cz_v, max_v):
    wid = lax.axis_index("s") * NC + lax.axis_index("c")
    base_row = wid * ROWS
    bigv = jnp.full((L,), BIG, jnp.int32)

    def row_body(r, maxcnt):
        @pl.when(r % GR == 0)
        def _fetch():
            pltpu.sync_copy(keys_hbm.at[pl.ds((base_row + r) * N, GR * N)],
                            kb_v)

        rb = (r % GR) * N
        for j in range(BUF // L):
            keys_v[pl.ds(j * L, L)] = bigv

        def chunk_body(i, cnt):
            kv = kb_v[pl.ds(rb + i * L, L)]
            hit = kv < BIG
            inc = hit.astype(jnp.int32)
            pre = plsc.cumsum(inc)
            idx = cnt + pre - 1
            plsc.store_scatter(keys_v, [idx], kv, mask=hit)
            return cnt + plsc.all_reduce_population_count(hit)

        cnt = lax.fori_loop(0, NCHUNK, chunk_body, jnp.zeros((L,), jnp.int32),
                            unroll=2)

        vs = [keys_v[pl.ds(j * L, L)] for j in range(BUF // L)]
        svs = _sort_vregs(vs, need=MAXN // L)
        for j in range(MAXN // L):
            k = svs[j]
            pad = k >= jnp.int32(27 * N)
            p = jnp.where(pad, -1, k & (N - 1))
            c = k >> 10
            ob = r * MAXN + j * L
            nbr_v[pl.ds(ob, L)] = p
            cx_v[pl.ds(ob, L)] = jnp.where(pad, 1, c % 3 - 1)
            cy_v[pl.ds(ob, L)] = jnp.where(pad, 1, (c // 3) % 3 - 1)
            cz_v[pl.ds(ob, L)] = jnp.where(pad, 1, c // 9 - 1)
        return jnp.maximum(maxcnt, cnt)

    maxcnt = lax.fori_loop(0, ROWS, row_body, jnp.zeros((L,), jnp.int32))
    max_v[...] = maxcnt
    span = pl.ds(wid * ROWS * MAXN, ROWS * MAXN)
    pltpu.sync_copy(nbr_v, nbr_hbm.at[span])
    pltpu.sync_copy(cx_v, cx_hbm.at[span])
    pltpu.sync_copy(cy_v, cy_hbm.at[span])
    pltpu.sync_copy(cz_v, cz_hbm.at[span])
    pltpu.sync_copy(max_v, max_hbm.at[wid])


def kernel(positions, max_neighbours):
    del max_neighbours
    pos = positions.astype(jnp.float32)
    posmat = pos.T.reshape(3, 8, 128)
    keys = _tc_keys(pos, posmat)
    nbr_flat, cx_flat, cy_flat, cz_flat, maxs = _sc_kernel(
        keys.reshape(N * N))
    neighbours = nbr_flat.reshape(N, MAXN)
    cells = jnp.stack(
        [cx_flat.reshape(N, MAXN), cy_flat.reshape(N, MAXN),
         cz_flat.reshape(N, MAXN)], axis=-1)
    return neighbours, cells, jnp.max(maxs)
